# baseline (speedup 1.0000x reference)
import jax
import jax.numpy as jnp
from jax.experimental import pallas as pl
from jax.experimental.pallas import tpu as pltpu

B = 1024
K = 2048
N = 100000
BN = 1024


def _matmul_block(x_ref, lut_ref, out_ref):
    xb = x_ref[...].astype(jnp.bfloat16)
    lb = lut_ref[...].astype(jnp.bfloat16)
    out_ref[...] = jax.lax.dot_general(
        xb, lb,
        dimension_numbers=(((1,), (1,)), ((), ())),
        preferred_element_type=jnp.float32,
    )


def kernel(x, person_id, LUT):
    del person_id
    grid = (pl.cdiv(N, BN),)
    return pl.pallas_call(
        _matmul_block,
        grid=grid,
        in_specs=[
            pl.BlockSpec((B, K), lambda i: (0, 0)),
            pl.BlockSpec((BN, K), lambda i: (i, 0)),
        ],
        out_specs=pl.BlockSpec((B, BN), lambda i: (0, i)),
        out_shape=jax.ShapeDtypeStruct((B, N), jnp.float32),
        compiler_params=pltpu.CompilerParams(
            dimension_semantics=("arbitrary",),
        ),
    )(x, LUT)
